# Initial kernel scaffold; baseline (speedup 1.0000x reference)
import jax
import jax.numpy as jnp
from jax.experimental import pallas as pl


def kernel(input, teacher_forcing):
    raise NotImplementedError("write your pallas kernel here")


# speedup vs baseline: 2.8115x; 2.8115x over previous
import jax
import jax.numpy as jnp
from jax.experimental import pallas as pl

DICT_SIZE = 1000


def _onehot_row_kernel(inp_ref, out_ref):
    idx = inp_ref[0, 0, :]
    b = idx.shape[0]
    d = jax.lax.broadcasted_iota(jnp.int32, (b, DICT_SIZE), 1)
    out_ref[0] = (d == idx[:, None]).astype(jnp.float32)


def kernel(input, teacher_forcing):
    S, B = input.shape
    inp3 = input.reshape(S, 1, B)
    p2 = pl.pallas_call(
        _onehot_row_kernel,
        grid=(S,),
        in_specs=[pl.BlockSpec((1, 1, B), lambda i: (i, 0, 0))],
        out_specs=pl.BlockSpec((1, B, DICT_SIZE), lambda i: (i, 0, 0)),
        out_shape=jax.ShapeDtypeStruct((S, B, DICT_SIZE), jnp.float32),
    )(inp3)
    return (p2, input)
